# baseline (speedup 1.0000x reference)
import jax
import jax.numpy as jnp
from jax import lax
from jax.experimental import pallas as pl
from jax.experimental.pallas import tpu as pltpu
from jax.experimental.pallas import tpu_sc as plsc

B, L, V, D = 1024, 200, 6, 128
NC, NS, LANES = 2, 16, 16
NW = NC * NS
LP = 256
NVEC = (L + LANES - 1) // LANES
GRP = (D // 8) * (LP // 128)

B_SC = 320
B_TC = B - B_SC
BPW = B_SC // NW

ROW_STRIDE = D + 1
REP_STRIDE = ((V * ROW_STRIDE + LANES) // LANES) * LANES + 1
TBL_WORDS = LANES * REP_STRIDE


def _sc_body(seq_hbm, tbl_hbm, out_hbm, seq_v, tbl_v, blk_v, sems):
    wid = lax.axis_index("s") * NC + lax.axis_index("c")
    base_b = wid * BPW
    pltpu.sync_copy(seq_hbm.at[pl.ds(base_b * L, BPW * L)], seq_v)
    pltpu.sync_copy(tbl_hbm, tbl_v)

    lanes = jnp.arange(LANES, dtype=jnp.int32)
    lane_base = lanes * REP_STRIDE

    def per_b(bi, carry):
        buf = lax.rem(bi, 2)

        @pl.when(bi >= 2)
        def _wait():
            pltpu.make_async_copy(
                blk_v.at[buf], out_hbm.at[base_b + bi - 2], sems.at[buf]
            ).wait()

        for j in range(NVEC):
            li = jnp.minimum(j * LANES + lanes, L - 1)
            seqv = plsc.load_gather(seq_v, [bi * L + li])
            addr0 = lane_base + seqv * ROW_STRIDE
            addrs0 = tuple(addr0 + k for k in range(16))
            jc = (j * LANES) // 128
            off = (j * LANES) % 128

            @plsc.parallel_loop(0, D, step=16, carry=addrs0)
            def _dloop(d0, addrs):
                row = (d0 // 8) * 2
                for k in range(16):
                    val = plsc.load_gather(tbl_v, [addrs[k]])
                    blk_v[buf, row + (k // 8) * 2 + jc, k % 8, pl.ds(off, LANES)] = val
                return tuple(a + 16 for a in addrs)

        pltpu.async_copy(blk_v.at[buf], out_hbm.at[base_b + bi], sems.at[buf])
        return carry

    lax.fori_loop(0, BPW, per_b, jnp.int32(0))
    for t in (BPW - 2, BPW - 1):
        pltpu.make_async_copy(
            blk_v.at[t % 2], out_hbm.at[base_b + t], sems.at[t % 2]
        ).wait()


def _tc_body(seq_ref, tblt_ref, out_ref):
    seq_row = seq_ref[0]
    onehot = (seq_row == jnp.arange(V, dtype=jnp.int32)[:, None]).astype(
        jnp.float32
    )
    out_ref[0] = jnp.dot(
        tblt_ref[...], onehot, preferred_element_type=jnp.float32
    )


def _cvt_body(prev_ref, sc_ref, out_ref):
    del prev_ref
    a = sc_ref[0, :, 0]
    out_ref[0, :, 0:128] = a.reshape(D, 128)
    b = sc_ref[0, :, 1].reshape(D, 128)
    out_ref[0, :, 128:L] = b[:, : L - 128]


@jax.jit
def kernel(seq, table):
    seq32 = seq.astype(jnp.int32)
    seq_sc = seq32[:B_SC].reshape(B_SC * L)
    row_pad = jnp.pad(table, ((0, 0), (0, ROW_STRIDE - D))).reshape(-1)
    rep = jnp.pad(row_pad, (0, REP_STRIDE - row_pad.shape[0]))
    tbl_flat = jnp.tile(rep, LANES)

    run_sc = pl.kernel(
        _sc_body,
        out_type=jax.ShapeDtypeStruct((B_SC, GRP, 8, 128), jnp.float32),
        mesh=plsc.VectorSubcoreMesh(core_axis_name="c", subcore_axis_name="s"),
        compiler_params=pltpu.CompilerParams(needs_layout_passes=False),
        scratch_types=[
            pltpu.VMEM((BPW * L,), jnp.int32),
            pltpu.VMEM((TBL_WORDS,), jnp.float32),
            pltpu.VMEM((2, GRP, 8, 128), jnp.float32),
            pltpu.SemaphoreType.DMA((2,)),
        ],
    )
    out_sc = run_sc(seq_sc, tbl_flat)

    seq_tc = seq32[B_SC:].reshape(B_TC, 1, L)
    run_tc = pl.pallas_call(
        _tc_body,
        grid=(B_TC,),
        in_specs=[
            pl.BlockSpec((1, 1, L), lambda g: (g, 0, 0)),
            pl.BlockSpec((D, V), lambda g: (0, 0)),
        ],
        out_specs=pl.BlockSpec((1, D, L), lambda g: (g + B_SC, 0, 0)),
        out_shape=jax.ShapeDtypeStruct((B, D, L), jnp.float32),
    )
    out_tc = run_tc(seq_tc, table.T)

    run_cvt = pl.pallas_call(
        _cvt_body,
        grid=(B_SC,),
        in_specs=[
            pl.BlockSpec(memory_space=pl.ANY),
            pl.BlockSpec((1, D // 8, 2, 8, 128), lambda g: (g, 0, 0, 0, 0)),
        ],
        out_specs=pl.BlockSpec((1, D, L), lambda g: (g, 0, 0)),
        out_shape=jax.ShapeDtypeStruct((B, D, L), jnp.float32),
        input_output_aliases={0: 0},
    )
    return run_cvt(out_tc, out_sc.reshape(B_SC, D // 8, 2, 8, 128))


# speedup vs baseline: 2.9671x; 2.9671x over previous
import jax
import jax.numpy as jnp
from jax import lax
from jax.experimental import pallas as pl
from jax.experimental.pallas import tpu as pltpu
from jax.experimental.pallas import tpu_sc as plsc

B, L, V, D = 1024, 200, 6, 128
NC, NS, LANES = 2, 16, 16
NW = NC * NS
LP = 256
NVEC = (L + LANES - 1) // LANES
GRP = (D // 8) * (LP // 128)

B_SC = 320
B_TC = B - B_SC
BPW = B_SC // NW

ROW_STRIDE = D + 1
REP_STRIDE = ((V * ROW_STRIDE + LANES) // LANES) * LANES + 1
TBL_WORDS = LANES * REP_STRIDE


def _sc_body(seq_hbm, tbl_hbm, out_hbm, seq_v, tbl_v, blk_v, sems):
    wid = lax.axis_index("s") * NC + lax.axis_index("c")
    base_b = wid * BPW
    pltpu.sync_copy(seq_hbm.at[pl.ds(base_b * L, BPW * L)], seq_v)
    pltpu.sync_copy(tbl_hbm, tbl_v)

    lanes = jnp.arange(LANES, dtype=jnp.int32)
    lane_base = lanes * REP_STRIDE

    def per_b(bi, carry):
        buf = lax.rem(bi, 2)

        @pl.when(bi >= 2)
        def _wait():
            pltpu.make_async_copy(
                blk_v.at[buf], out_hbm.at[base_b + bi - 2], sems.at[buf]
            ).wait()

        for j in range(NVEC):
            li = jnp.minimum(j * LANES + lanes, L - 1)
            seqv = plsc.load_gather(seq_v, [bi * L + li])
            addr0 = lane_base + seqv * ROW_STRIDE
            addrs0 = tuple(addr0 + k for k in range(16))
            jc = (j * LANES) // 128
            off = (j * LANES) % 128

            @plsc.parallel_loop(0, D, step=16, carry=addrs0)
            def _dloop(d0, addrs):
                row = (d0 // 8) * 2
                for k in range(16):
                    val = plsc.load_gather(tbl_v, [addrs[k]])
                    blk_v[buf, row + (k // 8) * 2 + jc, k % 8, pl.ds(off, LANES)] = val
                return tuple(a + 16 for a in addrs)

        pltpu.async_copy(blk_v.at[buf], out_hbm.at[base_b + bi], sems.at[buf])
        return carry

    lax.fori_loop(0, BPW, per_b, jnp.int32(0))
    for t in (BPW - 2, BPW - 1):
        pltpu.make_async_copy(
            blk_v.at[t % 2], out_hbm.at[base_b + t], sems.at[t % 2]
        ).wait()


BB_TC = 16
BB_CV = 8


def _tc_body(seq_ref, tblt_ref, out_ref):
    iv = jnp.arange(V, dtype=jnp.int32)[:, None]
    for bb in range(BB_TC):
        onehot = (seq_ref[bb] == iv).astype(jnp.float32)
        out_ref[bb] = jnp.dot(
            tblt_ref[...], onehot, preferred_element_type=jnp.float32
        )


def _cvt_body(prev_ref, sc_ref, out_ref):
    del prev_ref
    for bb in range(BB_CV):
        a = sc_ref[bb, :, 0]
        out_ref[bb, :, 0:128] = a.reshape(D, 128)
        b = sc_ref[bb, :, 1].reshape(D, 128)
        out_ref[bb, :, 128:L] = b[:, : L - 128]


@jax.jit
def kernel(seq, table):
    seq32 = seq.astype(jnp.int32)
    seq_sc = seq32[:B_SC].reshape(B_SC * L)
    row_pad = jnp.pad(table, ((0, 0), (0, ROW_STRIDE - D))).reshape(-1)
    rep = jnp.pad(row_pad, (0, REP_STRIDE - row_pad.shape[0]))
    tbl_flat = jnp.tile(rep, LANES)

    run_sc = pl.kernel(
        _sc_body,
        out_type=jax.ShapeDtypeStruct((B_SC, GRP, 8, 128), jnp.float32),
        mesh=plsc.VectorSubcoreMesh(core_axis_name="c", subcore_axis_name="s"),
        compiler_params=pltpu.CompilerParams(needs_layout_passes=False),
        scratch_types=[
            pltpu.VMEM((BPW * L,), jnp.int32),
            pltpu.VMEM((TBL_WORDS,), jnp.float32),
            pltpu.VMEM((2, GRP, 8, 128), jnp.float32),
            pltpu.SemaphoreType.DMA((2,)),
        ],
    )
    out_sc = run_sc(seq_sc, tbl_flat)

    seq_tc = seq32[B_SC:].reshape(B_TC, 1, L)
    run_tc = pl.pallas_call(
        _tc_body,
        grid=(B_TC // BB_TC,),
        in_specs=[
            pl.BlockSpec((BB_TC, 1, L), lambda g: (g, 0, 0)),
            pl.BlockSpec((D, V), lambda g: (0, 0)),
        ],
        out_specs=pl.BlockSpec(
            (BB_TC, D, L), lambda g: (g + B_SC // BB_TC, 0, 0)
        ),
        out_shape=jax.ShapeDtypeStruct((B, D, L), jnp.float32),
    )
    out_tc = run_tc(seq_tc, table.T)

    run_cvt = pl.pallas_call(
        _cvt_body,
        grid=(B_SC // BB_CV,),
        in_specs=[
            pl.BlockSpec(memory_space=pl.ANY),
            pl.BlockSpec(
                (BB_CV, D // 8, 2, 8, 128), lambda g: (g, 0, 0, 0, 0)
            ),
        ],
        out_specs=pl.BlockSpec((BB_CV, D, L), lambda g: (g, 0, 0)),
        out_shape=jax.ShapeDtypeStruct((B, D, L), jnp.float32),
        input_output_aliases={0: 0},
    )
    return run_cvt(out_tc, out_sc.reshape(B_SC, D // 8, 2, 8, 128))


# speedup vs baseline: 3.4999x; 1.1796x over previous
import jax
import jax.numpy as jnp
from jax import lax
from jax.experimental import pallas as pl
from jax.experimental.pallas import tpu as pltpu
from jax.experimental.pallas import tpu_sc as plsc

B, L, V, D = 1024, 200, 6, 128
NC, NS, LANES = 2, 16, 16
NW = NC * NS
LP = 256
NVEC = (L + LANES - 1) // LANES
GRP = (D // 8) * (LP // 128)

B_SC = 320
B_TC = B - B_SC
BPW = B_SC // NW

ROW_STRIDE = D + 1
REP_STRIDE = ((V * ROW_STRIDE + LANES) // LANES) * LANES + 1
TBL_WORDS = LANES * REP_STRIDE


def _sc_body(seq_hbm, tbl_hbm, out_hbm, seq_v, tbl_v, blk_v, sems):
    wid = lax.axis_index("s") * NC + lax.axis_index("c")
    base_b = wid * BPW
    pltpu.sync_copy(seq_hbm.at[pl.ds(base_b * L, BPW * L)], seq_v)
    pltpu.sync_copy(tbl_hbm, tbl_v)

    lanes = jnp.arange(LANES, dtype=jnp.int32)
    lane_base = lanes * REP_STRIDE

    def per_b(bi, carry):
        buf = lax.rem(bi, 2)

        @pl.when(bi >= 2)
        def _wait():
            pltpu.make_async_copy(
                blk_v.at[buf], out_hbm.at[base_b + bi - 2], sems.at[buf]
            ).wait()

        for j in range(NVEC):
            li = jnp.minimum(j * LANES + lanes, L - 1)
            seqv = plsc.load_gather(seq_v, [bi * L + li])
            addr0 = lane_base + seqv * ROW_STRIDE
            addrs0 = tuple(addr0 + k for k in range(16))
            jc = (j * LANES) // 128
            off = (j * LANES) % 128

            @plsc.parallel_loop(0, D, step=16, carry=addrs0)
            def _dloop(d0, addrs):
                row = (d0 // 8) * 2
                for k in range(16):
                    val = plsc.load_gather(tbl_v, [addrs[k]])
                    blk_v[buf, row + (k // 8) * 2 + jc, k % 8, pl.ds(off, LANES)] = val
                return tuple(a + 16 for a in addrs)

        pltpu.async_copy(blk_v.at[buf], out_hbm.at[base_b + bi], sems.at[buf])
        return carry

    lax.fori_loop(0, BPW, per_b, jnp.int32(0))
    for t in (BPW - 2, BPW - 1):
        pltpu.make_async_copy(
            blk_v.at[t % 2], out_hbm.at[base_b + t], sems.at[t % 2]
        ).wait()


BB_TC = 16
BB_CV = 8


def _tc_body(seq_ref, tblt_ref, out_ref):
    iv = jnp.arange(V, dtype=jnp.int32)[:, None]
    for bb in range(BB_TC):
        onehot = (seq_ref[bb] == iv).astype(jnp.float32)
        out_ref[bb] = jnp.dot(
            tblt_ref[...], onehot, preferred_element_type=jnp.float32
        )


def _cvt_body(prev_ref, sc_ref, out_ref):
    del prev_ref
    for bb in range(BB_CV):
        a = sc_ref[bb, :, 0]
        out_ref[bb, :, 0:128] = a.reshape(D, 128)
        b = sc_ref[bb, :, 1].reshape(D, 128)
        out_ref[bb, :, 128:L] = b[:, : L - 128]


@jax.jit
def kernel(seq, table):
    seq32 = seq.astype(jnp.int32)
    seq_sc = seq32[:B_SC].reshape(B_SC * L)
    row_pad = jnp.pad(table, ((0, 0), (0, ROW_STRIDE - D))).reshape(-1)
    rep = jnp.pad(row_pad, (0, REP_STRIDE - row_pad.shape[0]))
    tbl_flat = jnp.tile(rep, LANES)

    run_sc = pl.kernel(
        _sc_body,
        out_type=jax.ShapeDtypeStruct((B_SC, GRP, 8, 128), jnp.float32),
        mesh=plsc.VectorSubcoreMesh(core_axis_name="c", subcore_axis_name="s"),
        compiler_params=pltpu.CompilerParams(needs_layout_passes=False),
        scratch_types=[
            pltpu.VMEM((BPW * L,), jnp.int32),
            pltpu.VMEM((TBL_WORDS,), jnp.float32),
            pltpu.VMEM((2, GRP, 8, 128), jnp.float32),
            pltpu.SemaphoreType.DMA((2,)),
        ],
    )
    out_sc = run_sc(seq_sc, tbl_flat)

    seq_tc = seq32[B_SC:].reshape(B_TC, 1, L)
    run_tc = pl.pallas_call(
        _tc_body,
        grid=(B_TC // BB_TC,),
        in_specs=[
            pl.BlockSpec((BB_TC, 1, L), lambda g: (g, 0, 0)),
            pl.BlockSpec((D, V), lambda g: (0, 0)),
        ],
        out_specs=pl.BlockSpec(
            (BB_TC, D, L), lambda g: (g + B_SC // BB_TC, 0, 0)
        ),
        out_shape=jax.ShapeDtypeStruct((B, D, L), jnp.float32),
    )
    out_tc = run_tc(seq_tc, table.T)

    up = out_sc.reshape(B_SC, D // 8, 2, 8, 128)
    up = up.transpose(0, 1, 3, 2, 4).reshape(B_SC, D, LP)[:, :, :L]
    return lax.dynamic_update_slice(out_tc, up, (0, 0, 0))
